# Initial kernel scaffold; baseline (speedup 1.0000x reference)
import jax
import jax.numpy as jnp
from jax.experimental import pallas as pl


def kernel(z, codebook):
    raise NotImplementedError("write your pallas kernel here")


# speedup vs baseline: 1.0677x; 1.0677x over previous
import functools

import jax
import jax.numpy as jnp
from jax import lax
from jax.experimental import pallas as pl
from jax.experimental.pallas import tpu as pltpu
from jax.experimental.pallas import tpu_sc as plsc

BETA = 0.25

N = 8192
K = 8192
D = 64
BLK = 256
NGRID = N // BLK

NC = 2
NS = 16
NW = NC * NS
B_PER_W = N // NW
CHUNK = 128


CSEG = 4096
NSEG = K // CSEG


def _dist_argmin_body(zf_ref, zf2_ref, cb_ref, cb2_ref, idx_ref, loss_ref):
    i = pl.program_id(0)
    zfb = zf_ref[...]
    mm = lax.dot_general(zfb, cb_ref[...], (((1,), (1,)), ((), ())))
    d = (zf2_ref[...] - 2.0 * mm) + cb2_ref[...]
    acc_v = jnp.full((BLK, 1), jnp.inf, jnp.float32)
    acc_i = jnp.zeros((BLK, 1), jnp.int32)
    acc_d = jnp.zeros((BLK, 1), jnp.float32)
    for c in range(NSEG):
        dc = d[:, c * CSEG:(c + 1) * CSEG]
        m = jnp.min(dc, axis=1, keepdims=True)
        ii = lax.broadcasted_iota(jnp.int32, dc.shape, 1) + jnp.int32(c * CSEG)
        i_c = jnp.min(jnp.where(dc == m, ii, jnp.int32(K)), axis=1,
                      keepdims=True)
        keep = (acc_v < m) | ((acc_v == m) & (acc_i < i_c))
        acc_d = jnp.where(keep, acc_d, m)
        acc_i = jnp.where(keep, acc_i, i_c)
        acc_v = jnp.where(keep, acc_v, m).astype(jnp.bfloat16).astype(
            jnp.float32)
    idx_ref[0, 0, :] = acc_i[:, 0]

    @pl.when(i == 0)
    def _():
        loss_ref[...] = jnp.zeros_like(loss_ref)

    loss_ref[...] += jnp.sum(acc_d).reshape(1, 1)


def _dist_argmin(zf, zf2, cb, cb2):
    return pl.pallas_call(
        _dist_argmin_body,
        grid=(NGRID,),
        in_specs=[
            pl.BlockSpec((BLK, D), lambda i: (i, 0)),
            pl.BlockSpec((BLK, 1), lambda i: (i, 0)),
            pl.BlockSpec((K, D), lambda i: (0, 0)),
            pl.BlockSpec((1, K), lambda i: (0, 0)),
        ],
        out_specs=[
            pl.BlockSpec((1, 1, BLK), lambda i: (i, 0, 0)),
            pl.BlockSpec((1, 1), lambda i: (0, 0)),
        ],
        out_shape=[
            jax.ShapeDtypeStruct((NGRID, 1, BLK), jnp.int32),
            jax.ShapeDtypeStruct((1, 1), jnp.float32),
        ],
        compiler_params=pltpu.CompilerParams(
            dimension_semantics=("arbitrary",),
        ),
    )(zf, zf2, cb, cb2)


DPAD = 128


@functools.cache
def _make_sc_gather():
    @functools.partial(
        pl.kernel,
        mesh=plsc.VectorSubcoreMesh(core_axis_name="c", subcore_axis_name="s"),
        out_type=jax.ShapeDtypeStruct((N, DPAD), jnp.float32),
        scratch_types=[
            pltpu.VMEM((B_PER_W // CHUNK, CHUNK), jnp.int32),
            pltpu.VMEM((B_PER_W, DPAD), jnp.float32),
            pltpu.SemaphoreType.DMA,
        ],
    )
    def _sc_gather(table_hbm, idx_hbm, out_hbm, idx_v, rows_v, sem):
        wid = lax.axis_index("s") * NC + lax.axis_index("c")
        nrow = B_PER_W // CHUNK
        pltpu.sync_copy(idx_hbm.at[pl.ds(wid * nrow, nrow)], idx_v)
        copies = [
            pltpu.async_copy(
                table_hbm.at[idx_v.at[j]],
                rows_v.at[pl.ds(j * CHUNK, CHUNK)],
                sem,
            )
            for j in range(nrow)
        ]
        for c in copies:
            c.wait()
        pltpu.sync_copy(rows_v, out_hbm.at[pl.ds(wid * B_PER_W, B_PER_W)])

    return _sc_gather


def kernel(z, codebook):
    Bz, C, H, W = z.shape
    zf = jnp.transpose(z, (0, 2, 3, 1)).reshape(-1, C)
    zf2 = jnp.sum(zf * zf, axis=1, keepdims=True)
    cb2 = jnp.sum(codebook * codebook, axis=1)[None, :]

    idx3, loss_sum = _dist_argmin(zf, zf2, codebook, cb2)
    idx = idx3.reshape(N)

    cb_pad = jnp.pad(codebook, ((0, 0), (0, DPAD - D)))
    zq = _make_sc_gather()(cb_pad, idx.reshape(N // CHUNK, CHUNK))[:, :D]

    loss = (loss_sum[0, 0] / (N * D)) * (1.0 + BETA)
    zq_out = jnp.transpose(zq.reshape(Bz, H, W, C), (0, 3, 1, 2))
    return zq_out, loss, idx


# speedup vs baseline: 1.1287x; 1.0571x over previous
import functools

import jax
import jax.numpy as jnp
from jax import lax
from jax.experimental import pallas as pl
from jax.experimental.pallas import tpu as pltpu
from jax.experimental.pallas import tpu_sc as plsc

BETA = 0.25

N = 8192
K = 8192
D = 64
BLK = 512
NGRID = N // BLK

NC = 2
NS = 16
NW = NC * NS
B_PER_W = N // NW
CHUNK = 128


CSEG = 4096
NSEG = K // CSEG


def _dist_argmin_body(zf_ref, zf2_ref, cb_ref, cb2_ref, idx_ref, loss_ref):
    i = pl.program_id(0)
    zfb = zf_ref[...]
    mm = lax.dot_general(zfb, cb_ref[...], (((1,), (1,)), ((), ())))
    d = (zf2_ref[...] - 2.0 * mm) + cb2_ref[...]
    acc_v = jnp.full((BLK, 1), jnp.inf, jnp.float32)
    acc_i = jnp.zeros((BLK, 1), jnp.int32)
    acc_d = jnp.zeros((BLK, 1), jnp.float32)
    for c in range(NSEG):
        dc = d[:, c * CSEG:(c + 1) * CSEG]
        m = jnp.min(dc, axis=1, keepdims=True)
        ii = lax.broadcasted_iota(jnp.int32, dc.shape, 1) + jnp.int32(c * CSEG)
        i_c = jnp.min(jnp.where(dc == m, ii, jnp.int32(K)), axis=1,
                      keepdims=True)
        keep = (acc_v < m) | ((acc_v == m) & (acc_i < i_c))
        acc_d = jnp.where(keep, acc_d, m)
        acc_i = jnp.where(keep, acc_i, i_c)
        acc_v = jnp.where(keep, acc_v, m).astype(jnp.bfloat16).astype(
            jnp.float32)
    idx_ref[0, 0, :] = acc_i[:, 0]

    @pl.when(i == 0)
    def _():
        loss_ref[...] = jnp.zeros_like(loss_ref)

    loss_ref[...] += jnp.sum(acc_d).reshape(1, 1)


def _dist_argmin(zf, zf2, cb, cb2):
    return pl.pallas_call(
        _dist_argmin_body,
        grid=(NGRID,),
        in_specs=[
            pl.BlockSpec((BLK, D), lambda i: (i, 0)),
            pl.BlockSpec((BLK, 1), lambda i: (i, 0)),
            pl.BlockSpec((K, D), lambda i: (0, 0)),
            pl.BlockSpec((1, K), lambda i: (0, 0)),
        ],
        out_specs=[
            pl.BlockSpec((1, 1, BLK), lambda i: (i, 0, 0)),
            pl.BlockSpec((1, 1), lambda i: (0, 0)),
        ],
        out_shape=[
            jax.ShapeDtypeStruct((NGRID, 1, BLK), jnp.int32),
            jax.ShapeDtypeStruct((1, 1), jnp.float32),
        ],
        compiler_params=pltpu.CompilerParams(
            dimension_semantics=("arbitrary",),
        ),
    )(zf, zf2, cb, cb2)


DPAD = 128


@functools.cache
def _make_sc_gather():
    @functools.partial(
        pl.kernel,
        mesh=plsc.VectorSubcoreMesh(core_axis_name="c", subcore_axis_name="s"),
        out_type=jax.ShapeDtypeStruct((N, DPAD), jnp.float32),
        scratch_types=[
            pltpu.VMEM((B_PER_W // CHUNK, CHUNK), jnp.int32),
            pltpu.VMEM((B_PER_W, DPAD), jnp.float32),
            pltpu.SemaphoreType.DMA,
        ],
    )
    def _sc_gather(table_hbm, idx_hbm, out_hbm, idx_v, rows_v, sem):
        wid = lax.axis_index("s") * NC + lax.axis_index("c")
        nrow = B_PER_W // CHUNK
        pltpu.sync_copy(idx_hbm.at[pl.ds(wid * nrow, nrow)], idx_v)
        copies = [
            pltpu.async_copy(
                table_hbm.at[idx_v.at[j]],
                rows_v.at[pl.ds(j * CHUNK, CHUNK)],
                sem,
            )
            for j in range(nrow)
        ]
        for c in copies:
            c.wait()
        pltpu.sync_copy(rows_v, out_hbm.at[pl.ds(wid * B_PER_W, B_PER_W)])

    return _sc_gather


def kernel(z, codebook):
    Bz, C, H, W = z.shape
    zf = jnp.transpose(z, (0, 2, 3, 1)).reshape(-1, C)
    zf2 = jnp.sum(zf * zf, axis=1, keepdims=True)
    cb2 = jnp.sum(codebook * codebook, axis=1)[None, :]

    idx3, loss_sum = _dist_argmin(zf, zf2, codebook, cb2)
    idx = idx3.reshape(N)

    cb_pad = jnp.pad(codebook, ((0, 0), (0, DPAD - D)))
    zq = _make_sc_gather()(cb_pad, idx.reshape(N // CHUNK, CHUNK))[:, :D]

    loss = (loss_sum[0, 0] / (N * D)) * (1.0 + BETA)
    zq_out = jnp.transpose(zq.reshape(Bz, H, W, C), (0, 3, 1, 2))
    return zq_out, loss, idx


# speedup vs baseline: 1.1443x; 1.0138x over previous
import functools

import jax
import jax.numpy as jnp
from jax import lax
from jax.experimental import pallas as pl
from jax.experimental.pallas import tpu as pltpu
from jax.experimental.pallas import tpu_sc as plsc

BETA = 0.25

N = 8192
K = 8192
D = 64
BLK = 1024
NGRID = N // BLK

NC = 2
NS = 16
NW = NC * NS
B_PER_W = N // NW
CHUNK = 128


CSEG = 4096
NSEG = K // CSEG


def _dist_argmin_body(zf_ref, zf2_ref, cb_ref, cb2_ref, idx_ref, loss_ref):
    i = pl.program_id(0)
    zfb = zf_ref[...]
    mm = lax.dot_general(zfb, cb_ref[...], (((1,), (1,)), ((), ())))
    d = (zf2_ref[...] - 2.0 * mm) + cb2_ref[...]
    acc_v = jnp.full((BLK, 1), jnp.inf, jnp.float32)
    acc_i = jnp.zeros((BLK, 1), jnp.int32)
    acc_d = jnp.zeros((BLK, 1), jnp.float32)
    for c in range(NSEG):
        dc = d[:, c * CSEG:(c + 1) * CSEG]
        m = jnp.min(dc, axis=1, keepdims=True)
        ii = lax.broadcasted_iota(jnp.int32, dc.shape, 1) + jnp.int32(c * CSEG)
        i_c = jnp.min(jnp.where(dc == m, ii, jnp.int32(K)), axis=1,
                      keepdims=True)
        keep = (acc_v < m) | ((acc_v == m) & (acc_i < i_c))
        acc_d = jnp.where(keep, acc_d, m)
        acc_i = jnp.where(keep, acc_i, i_c)
        acc_v = jnp.where(keep, acc_v, m).astype(jnp.bfloat16).astype(
            jnp.float32)
    idx_ref[0, 0, :] = acc_i[:, 0]

    @pl.when(i == 0)
    def _():
        loss_ref[...] = jnp.zeros_like(loss_ref)

    loss_ref[...] += jnp.sum(acc_d).reshape(1, 1)


def _dist_argmin(zf, zf2, cb, cb2):
    return pl.pallas_call(
        _dist_argmin_body,
        grid=(NGRID,),
        in_specs=[
            pl.BlockSpec((BLK, D), lambda i: (i, 0)),
            pl.BlockSpec((BLK, 1), lambda i: (i, 0)),
            pl.BlockSpec((K, D), lambda i: (0, 0)),
            pl.BlockSpec((1, K), lambda i: (0, 0)),
        ],
        out_specs=[
            pl.BlockSpec((1, 1, BLK), lambda i: (i, 0, 0)),
            pl.BlockSpec((1, 1), lambda i: (0, 0)),
        ],
        out_shape=[
            jax.ShapeDtypeStruct((NGRID, 1, BLK), jnp.int32),
            jax.ShapeDtypeStruct((1, 1), jnp.float32),
        ],
        compiler_params=pltpu.CompilerParams(
            dimension_semantics=("arbitrary",),
        ),
    )(zf, zf2, cb, cb2)


DPAD = 128


@functools.cache
def _make_sc_gather():
    @functools.partial(
        pl.kernel,
        mesh=plsc.VectorSubcoreMesh(core_axis_name="c", subcore_axis_name="s"),
        out_type=jax.ShapeDtypeStruct((N, DPAD), jnp.float32),
        scratch_types=[
            pltpu.VMEM((B_PER_W // CHUNK, CHUNK), jnp.int32),
            pltpu.VMEM((B_PER_W, DPAD), jnp.float32),
            pltpu.SemaphoreType.DMA,
        ],
    )
    def _sc_gather(table_hbm, idx_hbm, out_hbm, idx_v, rows_v, sem):
        wid = lax.axis_index("s") * NC + lax.axis_index("c")
        nrow = B_PER_W // CHUNK
        pltpu.sync_copy(idx_hbm.at[pl.ds(wid * nrow, nrow)], idx_v)
        copies = [
            pltpu.async_copy(
                table_hbm.at[idx_v.at[j]],
                rows_v.at[pl.ds(j * CHUNK, CHUNK)],
                sem,
            )
            for j in range(nrow)
        ]
        for c in copies:
            c.wait()
        pltpu.sync_copy(rows_v, out_hbm.at[pl.ds(wid * B_PER_W, B_PER_W)])

    return _sc_gather


def kernel(z, codebook):
    Bz, C, H, W = z.shape
    zf = jnp.transpose(z, (0, 2, 3, 1)).reshape(-1, C)
    zf2 = jnp.sum(zf * zf, axis=1, keepdims=True)
    cb2 = jnp.sum(codebook * codebook, axis=1)[None, :]

    idx3, loss_sum = _dist_argmin(zf, zf2, codebook, cb2)
    idx = idx3.reshape(N)

    cb_pad = jnp.pad(codebook, ((0, 0), (0, DPAD - D)))
    zq = _make_sc_gather()(cb_pad, idx.reshape(N // CHUNK, CHUNK))[:, :D]

    loss = (loss_sum[0, 0] / (N * D)) * (1.0 + BETA)
    zq_out = jnp.transpose(zq.reshape(Bz, H, W, C), (0, 3, 1, 2))
    return zq_out, loss, idx
